# baseline (speedup 1.0000x reference)
import functools
import jax
import jax.numpy as jnp
import numpy as np
from jax import lax
from jax.experimental import pallas as pl
from jax.experimental.pallas import tpu as pltpu
from jax.experimental.pallas import tpu_sc as plsc

N = 10000
IN_DIM = 128
HID_DIM = 512
OUT_DIM = 768
E = 320000

NC = 2
NS = 16
NW = NC * NS
B = 88
NB = -(-E // (B * NW))
NB = ((NB + 7) // 8) * 8
TB = NB * NW
EP = TB * B
IDX_LEN = EP + 8 * B
NPAD = 10112
ROWS_PER_TILE = NPAD // NS

BN = 512
GRID = -(-NPAD // BN)

_AR = np.arange(IDX_LEN - E, dtype=np.int32)
_PAD_SRC = (N + _AR % (NPAD - N)).astype(np.int32)
_PAD_DST = (_AR % NPAD).astype(np.int32)
_PAD_DEG = (N + _AR % (NPAD - N)).astype(np.int32)

@functools.cache
def _mesh():
    return plsc.VectorSubcoreMesh(core_axis_name="c", subcore_axis_name="s",
                                  num_cores=NC, num_subcores=NS)


def _deg_body(zeros_hbm, ones_hbm, dst_hbm, out_hbm, di0, di1, rows, acc,
              is0, is1):
    c = lax.axis_index("c")
    s = lax.axis_index("s")
    wid = s * NC + c
    r0 = s * ROWS_PER_TILE
    pltpu.sync_copy(zeros_hbm.at[pl.ds(r0, ROWS_PER_TILE)],
                    acc.at[pl.ds(r0, ROWS_PER_TILE)])
    base = wid * (NB * B)
    pltpu.async_copy(dst_hbm.at[pl.ds(base, B)], di0, is0)
    pltpu.async_copy(dst_hbm.at[pl.ds(base + B, B)], di1, is1)
    pltpu.sync_copy(ones_hbm, rows)
    plsc.subcore_barrier()

    @pl.loop(0, NB // 2)
    def _(j):
        b = base + 2 * j * B
        pltpu.make_async_copy(dst_hbm.at[pl.ds(0, B)], di0, is0).wait()
        pltpu.sync_copy(rows, acc.at[di0], add=True)
        pltpu.async_copy(dst_hbm.at[pl.ds(b + 2 * B, B)], di0, is0)
        pltpu.make_async_copy(dst_hbm.at[pl.ds(0, B)], di1, is1).wait()
        pltpu.sync_copy(rows, acc.at[di1], add=True)
        pltpu.async_copy(dst_hbm.at[pl.ds(b + 3 * B, B)], di1, is1)

    pltpu.make_async_copy(dst_hbm.at[pl.ds(0, B)], di0, is0).wait()
    pltpu.make_async_copy(dst_hbm.at[pl.ds(0, B)], di1, is1).wait()
    plsc.subcore_barrier()
    pltpu.sync_copy(acc.at[pl.ds(r0, ROWS_PER_TILE)],
                    out_hbm.at[c, pl.ds(r0, ROWS_PER_TILE)])


def _prop_body(zeros_hbm, src_hbm, dst_hbm, table_hbm, out_hbm, *sc):
    si = sc[0:8]
    di = sc[8:16]
    rows = sc[16:20]
    acc = sc[20]
    gs = sc[21:25]
    ss = sc[25:29]
    isem = sc[29:37]
    c = lax.axis_index("c")
    s = lax.axis_index("s")
    wid = s * NC + c
    r0 = s * ROWS_PER_TILE
    pltpu.sync_copy(zeros_hbm.at[pl.ds(r0, ROWS_PER_TILE)],
                    acc.at[pl.ds(r0, ROWS_PER_TILE)])
    base = wid * NB * B

    def idx_load(lb, k8, sync=False):
        off = base + lb * B
        if sync:
            pltpu.sync_copy(src_hbm.at[pl.ds(off, B)], si[k8])
            pltpu.sync_copy(dst_hbm.at[pl.ds(off, B)], di[k8])
        else:
            pltpu.async_copy(src_hbm.at[pl.ds(off, B)], si[k8], isem[k8])
            pltpu.async_copy(dst_hbm.at[pl.ds(off, B)], di[k8], isem[k8])

    def idx_wait(t8):
        pltpu.make_async_copy(src_hbm.at[pl.ds(0, B)], si[t8], isem[t8]).wait()
        pltpu.make_async_copy(dst_hbm.at[pl.ds(0, B)], di[t8], isem[t8]).wait()

    def gather_start(lb, t4, t8):
        pltpu.async_copy(table_hbm.at[si[t8]], rows[t4], gs[t4])

    def gather_wait(t4, t8):
        pltpu.make_async_copy(table_hbm.at[si[t8]], rows[t4], gs[t4]).wait()

    def scatter_start(t4, t8):
        pltpu.async_copy(rows[t4], acc.at[di[t8]], ss[t4], add=True)

    def scatter_wait(t4, t8):
        pltpu.make_async_copy(rows[t4], acc.at[di[t8]], ss[t4]).wait()

    def stage(lb, t, skip_ss=False):
        gather_wait(t % 4, t % 8)
        scatter_start(t % 4, t % 8)
        if not skip_ss:
            scatter_wait((t + 2) % 4, (t + 2) % 8)
        idx_load(lb + 6, (t + 6) % 8)
        idx_wait((t + 2) % 8)
        gather_start(lb + 2, (t + 2) % 4, (t + 2) % 8)

    idx_load(0, 0, sync=True)
    idx_load(1, 1, sync=True)
    for t in range(2, 6):
        idx_load(t, t)
    plsc.subcore_barrier()
    gather_start(0, 0, 0)
    gather_start(1, 1, 1)
    for t in range(8):
        stage(t, t, skip_ss=(t < 2))

    @pl.loop(1, NB // 8)
    def _(j):
        b0 = j * 8
        for t in range(8):
            stage(b0 + t, t)

    gather_wait(0, 0)
    gather_wait(1, 1)
    scatter_wait(2, 2)
    scatter_wait(3, 3)
    for t in range(2, 6):
        idx_wait(t)
    plsc.subcore_barrier()
    pltpu.sync_copy(acc.at[pl.ds(r0, ROWS_PER_TILE)],
                    out_hbm.at[c, pl.ds(r0, ROWS_PER_TILE)])


@functools.cache
def _make_deg(interpret=False):
    return pl.kernel(
        _deg_body,
        out_type=jax.ShapeDtypeStruct((NC, NPAD, IN_DIM), jnp.float32),
        mesh=_mesh(),
        scratch_types=[
            pltpu.VMEM((B,), jnp.int32),
            pltpu.VMEM((B,), jnp.int32),
            pltpu.VMEM((B, IN_DIM), jnp.float32),
            pltpu.VMEM_SHARED((NPAD, IN_DIM), jnp.float32),
            pltpu.SemaphoreType.DMA,
            pltpu.SemaphoreType.DMA,
        ],
        interpret=interpret,
    )


@functools.cache
def _make_prop(interpret=False):
    return pl.kernel(
        _prop_body,
        out_type=jax.ShapeDtypeStruct((NC, NPAD, IN_DIM), jnp.float32),
        mesh=_mesh(),
        scratch_types=(
            [pltpu.VMEM((B,), jnp.int32)] * 16
            + [pltpu.VMEM((B, IN_DIM), jnp.float32)] * 4
            + [pltpu.VMEM_SHARED((NPAD, IN_DIM), jnp.float32)]
            + [pltpu.SemaphoreType.DMA] * 16
        ),
        interpret=interpret,
    )


def _scale_body(deg2_ref, x_ref, y1_ref, dinv_ref):
    deg = deg2_ref[0, :, 0:1] + deg2_ref[1, :, 0:1] + 1.0
    dinv = lax.rsqrt(deg)
    y1_ref[...] = x_ref[...] * dinv
    dinv_ref[...] = jnp.broadcast_to(dinv, (BN, IN_DIM))


def _make_scale(interpret=False):
    return pl.pallas_call(
        _scale_body,
        grid=(GRID,),
        in_specs=[
            pl.BlockSpec((NC, BN, IN_DIM), lambda i: (0, i, 0)),
            pl.BlockSpec((BN, IN_DIM), lambda i: (i, 0)),
        ],
        out_specs=[
            pl.BlockSpec((BN, IN_DIM), lambda i: (i, 0)),
            pl.BlockSpec((BN, IN_DIM), lambda i: (i, 0)),
        ],
        out_shape=[
            jax.ShapeDtypeStruct((NPAD, IN_DIM), jnp.float32),
            jax.ShapeDtypeStruct((NPAD, IN_DIM), jnp.float32),
        ],
        interpret=interpret,
    )


def _mm1_body(acc_ref, y1_ref, dinv_ref, w1_ref, b1_ref, *y2_refs):
    dinv = dinv_ref[...]
    sx = (acc_ref[0] + acc_ref[1] + y1_ref[...]) * dinv
    h = jnp.dot(sx, w1_ref[...], preferred_element_type=jnp.float32,
                precision=lax.Precision.DEFAULT)
    h = jnp.maximum(h + b1_ref[...], 0.0)
    y2 = h * dinv[:, 0:1]
    for ck in range(4):
        y2_refs[ck][...] = y2[:, ck * IN_DIM:(ck + 1) * IN_DIM]


def _make_mm1(interpret=False):
    return pl.pallas_call(
        _mm1_body,
        grid=(GRID,),
        in_specs=[
            pl.BlockSpec((NC, BN, IN_DIM), lambda i: (0, i, 0)),
            pl.BlockSpec((BN, IN_DIM), lambda i: (i, 0)),
            pl.BlockSpec((BN, IN_DIM), lambda i: (i, 0)),
            pl.BlockSpec((IN_DIM, HID_DIM), lambda i: (0, 0)),
            pl.BlockSpec((1, HID_DIM), lambda i: (0, 0)),
        ],
        out_specs=[pl.BlockSpec((BN, IN_DIM), lambda i: (i, 0))] * 4,
        out_shape=[jax.ShapeDtypeStruct((NPAD, IN_DIM), jnp.float32)] * 4,
        interpret=interpret,
    )


def _mm2_body(a0_ref, a1_ref, a2_ref, a3_ref, y0_ref, y1c_ref, y2c_ref,
              y3_ref, dinv_ref, x_ref, w2_ref, wl_ref, b2_ref, bl_ref,
              out_ref):
    dinv = dinv_ref[...]
    accs = (a0_ref, a1_ref, a2_ref, a3_ref)
    ys = (y0_ref, y1c_ref, y2c_ref, y3_ref)
    h2 = jnp.broadcast_to(b2_ref[...], (BN, OUT_DIM))
    for ck in range(4):
        sx = (accs[ck][0] + accs[ck][1] + ys[ck][...]) * dinv
        h2 = h2 + jnp.dot(sx, w2_ref[ck * IN_DIM:(ck + 1) * IN_DIM, :],
                          preferred_element_type=jnp.float32,
                          precision=lax.Precision.DEFAULT)
    out = jnp.dot(h2, wl_ref[...], preferred_element_type=jnp.float32,
                  precision=lax.Precision.DEFAULT)
    out = jnp.maximum(out + bl_ref[...], 0.0)
    out_ref[:, 0:IN_DIM] = x_ref[...]
    out_ref[:, IN_DIM:] = out


def _make_mm2(interpret=False):
    return pl.pallas_call(
        _mm2_body,
        grid=(GRID,),
        in_specs=[
            pl.BlockSpec((NC, BN, IN_DIM), lambda i: (0, i, 0)),
            pl.BlockSpec((NC, BN, IN_DIM), lambda i: (0, i, 0)),
            pl.BlockSpec((NC, BN, IN_DIM), lambda i: (0, i, 0)),
            pl.BlockSpec((NC, BN, IN_DIM), lambda i: (0, i, 0)),
            pl.BlockSpec((BN, IN_DIM), lambda i: (i, 0)),
            pl.BlockSpec((BN, IN_DIM), lambda i: (i, 0)),
            pl.BlockSpec((BN, IN_DIM), lambda i: (i, 0)),
            pl.BlockSpec((BN, IN_DIM), lambda i: (i, 0)),
            pl.BlockSpec((BN, IN_DIM), lambda i: (i, 0)),
            pl.BlockSpec((BN, IN_DIM), lambda i: (i, 0)),
            pl.BlockSpec((HID_DIM, OUT_DIM), lambda i: (0, 0)),
            pl.BlockSpec((OUT_DIM, OUT_DIM), lambda i: (0, 0)),
            pl.BlockSpec((1, OUT_DIM), lambda i: (0, 0)),
            pl.BlockSpec((1, OUT_DIM), lambda i: (0, 0)),
        ],
        out_specs=pl.BlockSpec((BN, IN_DIM + OUT_DIM), lambda i: (i, 0)),
        out_shape=jax.ShapeDtypeStruct((N, IN_DIM + OUT_DIM), jnp.float32),
        interpret=interpret,
    )


_scale_k = _make_scale()
_mm1_k = _make_mm1()
_mm2_k = _make_mm2()


@jax.jit
def kernel(feature, edge_index, W1, b1, W2, b2, Wl, bl):
    _deg_k = _make_deg()
    _prop_k = _make_prop()
    src = edge_index[0].astype(jnp.int32)
    dst = edge_index[1].astype(jnp.int32)
    srcp = jnp.concatenate([src, jnp.asarray(_PAD_SRC)])
    dstp = jnp.concatenate([dst, jnp.asarray(_PAD_DST)])
    dstp_deg = jnp.concatenate([dst, jnp.asarray(_PAD_DEG)])
    xpad = jnp.pad(feature, ((0, NPAD - N), (0, 0)))
    zeros128 = jnp.zeros((NPAD, IN_DIM), jnp.float32)
    ones128 = jnp.ones((B, IN_DIM), jnp.float32)

    deg2 = _deg_k(zeros128, ones128, dstp_deg)
    y1, dinv128 = _scale_k(deg2, xpad)
    acc1 = _prop_k(zeros128, srcp, dstp, y1)
    y2_4 = _mm1_k(acc1, y1, dinv128, W1, b1.reshape(1, HID_DIM))
    accs = [_prop_k(zeros128, srcp, dstp, y2_4[ck]) for ck in range(4)]
    out = _mm2_k(accs[0], accs[1], accs[2], accs[3], *y2_4, dinv128, xpad,
                 W2, Wl, b2.reshape(1, OUT_DIM), bl.reshape(1, OUT_DIM))
    return out[:N]


# speedup vs baseline: 1.0935x; 1.0935x over previous
import functools
import jax
import jax.numpy as jnp
import numpy as np
from jax import lax
from jax.experimental import pallas as pl
from jax.experimental.pallas import tpu as pltpu
from jax.experimental.pallas import tpu_sc as plsc

N = 10000
IN_DIM = 128
HID_DIM = 512
OUT_DIM = 768
E = 320000

NC = 2
NS = 16
NW = NC * NS
B = 128
NB = -(-E // (B * NW))
NB = ((NB + 3) // 4) * 4
TB = NB * NW
EP = TB * B
IDX_LEN = EP + 8 * B
NPAD = 10112
ROWS_PER_TILE = NPAD // NS

BN = 1024
GRID = -(-NPAD // BN)

_AR = np.arange(IDX_LEN - E, dtype=np.int32)
_PAD_SRC = (N + _AR % (NPAD - N)).astype(np.int32)
_PAD_DST = (_AR % NPAD).astype(np.int32)
_PAD_DEG = (N + _AR % (NPAD - N)).astype(np.int32)

@functools.cache
def _mesh():
    return plsc.VectorSubcoreMesh(core_axis_name="c", subcore_axis_name="s",
                                  num_cores=NC, num_subcores=NS)


def _deg_body(zeros_hbm, ones_hbm, dst_hbm, out_hbm, di0, di1, rows, acc,
              is0, is1):
    c = lax.axis_index("c")
    s = lax.axis_index("s")
    wid = s * NC + c
    r0 = s * ROWS_PER_TILE
    pltpu.sync_copy(zeros_hbm.at[pl.ds(r0, ROWS_PER_TILE)],
                    acc.at[pl.ds(r0, ROWS_PER_TILE)])
    base = wid * (NB * B)
    pltpu.async_copy(dst_hbm.at[pl.ds(base, B)], di0, is0)
    pltpu.async_copy(dst_hbm.at[pl.ds(base + B, B)], di1, is1)
    pltpu.sync_copy(ones_hbm, rows)
    plsc.subcore_barrier()

    @pl.loop(0, NB // 2)
    def _(j):
        b = base + 2 * j * B
        pltpu.make_async_copy(dst_hbm.at[pl.ds(0, B)], di0, is0).wait()
        pltpu.sync_copy(rows, acc.at[di0], add=True)
        pltpu.async_copy(dst_hbm.at[pl.ds(b + 2 * B, B)], di0, is0)
        pltpu.make_async_copy(dst_hbm.at[pl.ds(0, B)], di1, is1).wait()
        pltpu.sync_copy(rows, acc.at[di1], add=True)
        pltpu.async_copy(dst_hbm.at[pl.ds(b + 3 * B, B)], di1, is1)

    pltpu.make_async_copy(dst_hbm.at[pl.ds(0, B)], di0, is0).wait()
    pltpu.make_async_copy(dst_hbm.at[pl.ds(0, B)], di1, is1).wait()
    plsc.subcore_barrier()
    pltpu.sync_copy(acc.at[pl.ds(r0, ROWS_PER_TILE)],
                    out_hbm.at[c, pl.ds(r0, ROWS_PER_TILE)])


def _prop_body(zeros_hbm, src_hbm, dst_hbm, table_hbm, out_hbm,
               si0, di0, si1, di1, si2, di2, si3, di3, rows0, rows1, acc,
               gs0, gs1, is0, is1, is2, is3):
    c = lax.axis_index("c")
    s = lax.axis_index("s")
    wid = s * NC + c
    r0 = s * ROWS_PER_TILE
    pltpu.sync_copy(zeros_hbm.at[pl.ds(r0, ROWS_PER_TILE)],
                    acc.at[pl.ds(r0, ROWS_PER_TILE)])
    base = wid * NB * B
    pltpu.sync_copy(src_hbm.at[pl.ds(base, B)], si0)
    pltpu.sync_copy(dst_hbm.at[pl.ds(base, B)], di0)
    pltpu.sync_copy(src_hbm.at[pl.ds(base + B, B)], si1)
    pltpu.sync_copy(dst_hbm.at[pl.ds(base + B, B)], di1)
    pltpu.async_copy(src_hbm.at[pl.ds(base + 2 * B, B)], si2, is2)
    pltpu.async_copy(dst_hbm.at[pl.ds(base + 2 * B, B)], di2, is2)
    pltpu.async_copy(src_hbm.at[pl.ds(base + 3 * B, B)], si3, is3)
    pltpu.async_copy(dst_hbm.at[pl.ds(base + 3 * B, B)], di3, is3)
    plsc.subcore_barrier()
    pltpu.async_copy(table_hbm.at[si0], rows0, gs0)
    pltpu.async_copy(table_hbm.at[si1], rows1, gs1)

    @pl.loop(0, NB // 4)
    def _(j):
        b = base + 4 * j * B
        pltpu.make_async_copy(table_hbm.at[si0], rows0, gs0).wait()
        pltpu.sync_copy(rows0, acc.at[di0], add=True)
        pltpu.async_copy(src_hbm.at[pl.ds(b + 4 * B, B)], si0, is0)
        pltpu.async_copy(dst_hbm.at[pl.ds(b + 4 * B, B)], di0, is0)
        pltpu.make_async_copy(src_hbm.at[pl.ds(0, B)], si2, is2).wait()
        pltpu.make_async_copy(dst_hbm.at[pl.ds(0, B)], di2, is2).wait()
        pltpu.async_copy(table_hbm.at[si2], rows0, gs0)
        pltpu.make_async_copy(table_hbm.at[si1], rows1, gs1).wait()
        pltpu.sync_copy(rows1, acc.at[di1], add=True)
        pltpu.async_copy(src_hbm.at[pl.ds(b + 5 * B, B)], si1, is1)
        pltpu.async_copy(dst_hbm.at[pl.ds(b + 5 * B, B)], di1, is1)
        pltpu.make_async_copy(src_hbm.at[pl.ds(0, B)], si3, is3).wait()
        pltpu.make_async_copy(dst_hbm.at[pl.ds(0, B)], di3, is3).wait()
        pltpu.async_copy(table_hbm.at[si3], rows1, gs1)
        pltpu.make_async_copy(table_hbm.at[si2], rows0, gs0).wait()
        pltpu.sync_copy(rows0, acc.at[di2], add=True)
        pltpu.async_copy(src_hbm.at[pl.ds(b + 6 * B, B)], si2, is2)
        pltpu.async_copy(dst_hbm.at[pl.ds(b + 6 * B, B)], di2, is2)
        pltpu.make_async_copy(src_hbm.at[pl.ds(0, B)], si0, is0).wait()
        pltpu.make_async_copy(dst_hbm.at[pl.ds(0, B)], di0, is0).wait()
        pltpu.async_copy(table_hbm.at[si0], rows0, gs0)
        pltpu.make_async_copy(table_hbm.at[si1], rows1, gs1).wait()
        pltpu.sync_copy(rows1, acc.at[di3], add=True)
        pltpu.async_copy(src_hbm.at[pl.ds(b + 7 * B, B)], si3, is3)
        pltpu.async_copy(dst_hbm.at[pl.ds(b + 7 * B, B)], di3, is3)
        pltpu.make_async_copy(src_hbm.at[pl.ds(0, B)], si1, is1).wait()
        pltpu.make_async_copy(dst_hbm.at[pl.ds(0, B)], di1, is1).wait()
        pltpu.async_copy(table_hbm.at[si1], rows1, gs1)

    pltpu.make_async_copy(table_hbm.at[si0], rows0, gs0).wait()
    pltpu.make_async_copy(table_hbm.at[si1], rows1, gs1).wait()
    pltpu.make_async_copy(src_hbm.at[pl.ds(0, B)], si2, is2).wait()
    pltpu.make_async_copy(dst_hbm.at[pl.ds(0, B)], di2, is2).wait()
    pltpu.make_async_copy(src_hbm.at[pl.ds(0, B)], si3, is3).wait()
    pltpu.make_async_copy(dst_hbm.at[pl.ds(0, B)], di3, is3).wait()
    plsc.subcore_barrier()
    pltpu.sync_copy(acc.at[pl.ds(r0, ROWS_PER_TILE)],
                    out_hbm.at[c, pl.ds(r0, ROWS_PER_TILE)])


@functools.cache
def _make_deg(interpret=False):
    return pl.kernel(
        _deg_body,
        out_type=jax.ShapeDtypeStruct((NC, NPAD, IN_DIM), jnp.float32),
        mesh=_mesh(),
        scratch_types=[
            pltpu.VMEM((B,), jnp.int32),
            pltpu.VMEM((B,), jnp.int32),
            pltpu.VMEM((B, IN_DIM), jnp.float32),
            pltpu.VMEM_SHARED((NPAD, IN_DIM), jnp.float32),
            pltpu.SemaphoreType.DMA,
            pltpu.SemaphoreType.DMA,
        ],
        interpret=interpret,
    )


@functools.cache
def _make_prop(interpret=False):
    return pl.kernel(
        _prop_body,
        out_type=jax.ShapeDtypeStruct((NC, NPAD, IN_DIM), jnp.float32),
        mesh=_mesh(),
        scratch_types=(
            [pltpu.VMEM((B,), jnp.int32)] * 8
            + [pltpu.VMEM((B, IN_DIM), jnp.float32)] * 2
            + [pltpu.VMEM_SHARED((NPAD, IN_DIM), jnp.float32)]
            + [pltpu.SemaphoreType.DMA] * 6
        ),
        interpret=interpret,
    )


def _scale_body(deg2_ref, x_ref, y1_ref, dinv_ref):
    deg = deg2_ref[0, :, 0:1] + deg2_ref[1, :, 0:1] + 1.0
    dinv = lax.rsqrt(deg)
    y1_ref[...] = x_ref[...] * dinv
    dinv_ref[...] = jnp.broadcast_to(dinv, (BN, IN_DIM))


def _make_scale(interpret=False):
    return pl.pallas_call(
        _scale_body,
        grid=(GRID,),
        in_specs=[
            pl.BlockSpec((NC, BN, IN_DIM), lambda i: (0, i, 0)),
            pl.BlockSpec((BN, IN_DIM), lambda i: (i, 0)),
        ],
        out_specs=[
            pl.BlockSpec((BN, IN_DIM), lambda i: (i, 0)),
            pl.BlockSpec((BN, IN_DIM), lambda i: (i, 0)),
        ],
        out_shape=[
            jax.ShapeDtypeStruct((NPAD, IN_DIM), jnp.float32),
            jax.ShapeDtypeStruct((NPAD, IN_DIM), jnp.float32),
        ],
        interpret=interpret,
    )


def _mm1_body(acc_ref, y1_ref, dinv_ref, w1_ref, b1_ref, *y2_refs):
    dinv = dinv_ref[...]
    sx = (acc_ref[0] + acc_ref[1] + y1_ref[...]) * dinv
    h = jnp.dot(sx, w1_ref[...], preferred_element_type=jnp.float32,
                precision=lax.Precision.DEFAULT)
    h = jnp.maximum(h + b1_ref[...], 0.0)
    y2 = h * dinv[:, 0:1]
    for ck in range(4):
        y2_refs[ck][...] = y2[:, ck * IN_DIM:(ck + 1) * IN_DIM]


def _make_mm1(interpret=False):
    return pl.pallas_call(
        _mm1_body,
        grid=(GRID,),
        in_specs=[
            pl.BlockSpec((NC, BN, IN_DIM), lambda i: (0, i, 0)),
            pl.BlockSpec((BN, IN_DIM), lambda i: (i, 0)),
            pl.BlockSpec((BN, IN_DIM), lambda i: (i, 0)),
            pl.BlockSpec((IN_DIM, HID_DIM), lambda i: (0, 0)),
            pl.BlockSpec((1, HID_DIM), lambda i: (0, 0)),
        ],
        out_specs=[pl.BlockSpec((BN, IN_DIM), lambda i: (i, 0))] * 4,
        out_shape=[jax.ShapeDtypeStruct((NPAD, IN_DIM), jnp.float32)] * 4,
        interpret=interpret,
    )


def _mm2_body(a0_ref, a1_ref, a2_ref, a3_ref, y0_ref, y1c_ref, y2c_ref,
              y3_ref, dinv_ref, x_ref, w2_ref, wl_ref, b2_ref, bl_ref,
              out_ref):
    dinv = dinv_ref[...]
    accs = (a0_ref, a1_ref, a2_ref, a3_ref)
    ys = (y0_ref, y1c_ref, y2c_ref, y3_ref)
    h2 = jnp.broadcast_to(b2_ref[...], (BN, OUT_DIM))
    for ck in range(4):
        sx = (accs[ck][0] + accs[ck][1] + ys[ck][...]) * dinv
        h2 = h2 + jnp.dot(sx, w2_ref[ck * IN_DIM:(ck + 1) * IN_DIM, :],
                          preferred_element_type=jnp.float32,
                          precision=lax.Precision.DEFAULT)
    out = jnp.dot(h2, wl_ref[...], preferred_element_type=jnp.float32,
                  precision=lax.Precision.DEFAULT)
    out = jnp.maximum(out + bl_ref[...], 0.0)
    out_ref[:, 0:IN_DIM] = x_ref[...]
    out_ref[:, IN_DIM:] = out


def _make_mm2(interpret=False):
    return pl.pallas_call(
        _mm2_body,
        grid=(GRID,),
        in_specs=[
            pl.BlockSpec((NC, BN, IN_DIM), lambda i: (0, i, 0)),
            pl.BlockSpec((NC, BN, IN_DIM), lambda i: (0, i, 0)),
            pl.BlockSpec((NC, BN, IN_DIM), lambda i: (0, i, 0)),
            pl.BlockSpec((NC, BN, IN_DIM), lambda i: (0, i, 0)),
            pl.BlockSpec((BN, IN_DIM), lambda i: (i, 0)),
            pl.BlockSpec((BN, IN_DIM), lambda i: (i, 0)),
            pl.BlockSpec((BN, IN_DIM), lambda i: (i, 0)),
            pl.BlockSpec((BN, IN_DIM), lambda i: (i, 0)),
            pl.BlockSpec((BN, IN_DIM), lambda i: (i, 0)),
            pl.BlockSpec((BN, IN_DIM), lambda i: (i, 0)),
            pl.BlockSpec((HID_DIM, OUT_DIM), lambda i: (0, 0)),
            pl.BlockSpec((OUT_DIM, OUT_DIM), lambda i: (0, 0)),
            pl.BlockSpec((1, OUT_DIM), lambda i: (0, 0)),
            pl.BlockSpec((1, OUT_DIM), lambda i: (0, 0)),
        ],
        out_specs=pl.BlockSpec((BN, IN_DIM + OUT_DIM), lambda i: (i, 0)),
        out_shape=jax.ShapeDtypeStruct((N, IN_DIM + OUT_DIM), jnp.float32),
        interpret=interpret,
    )


_scale_k = _make_scale()
_mm1_k = _make_mm1()
_mm2_k = _make_mm2()


@jax.jit
def kernel(feature, edge_index, W1, b1, W2, b2, Wl, bl):
    _deg_k = _make_deg()
    _prop_k = _make_prop()
    src = edge_index[0].astype(jnp.int32)
    dst = edge_index[1].astype(jnp.int32)
    srcp = jnp.concatenate([src, jnp.asarray(_PAD_SRC)])
    dstp = jnp.concatenate([dst, jnp.asarray(_PAD_DST)])
    dstp_deg = jnp.concatenate([dst, jnp.asarray(_PAD_DEG)])
    xpad = jnp.pad(feature, ((0, NPAD - N), (0, 0)))
    zeros128 = jnp.zeros((NPAD, IN_DIM), jnp.float32)
    ones128 = jnp.ones((B, IN_DIM), jnp.float32)

    deg2 = _deg_k(zeros128, ones128, dstp_deg)
    y1, dinv128 = _scale_k(deg2, xpad)
    acc1 = _prop_k(zeros128, srcp, dstp, y1)
    y2_4 = _mm1_k(acc1, y1, dinv128, W1, b1.reshape(1, HID_DIM))
    accs = [_prop_k(zeros128, srcp, dstp, y2_4[ck]) for ck in range(4)]
    out = _mm2_k(accs[0], accs[1], accs[2], accs[3], *y2_4, dinv128, xpad,
                 W2, Wl, b2.reshape(1, OUT_DIM), bl.reshape(1, OUT_DIM))
    return out[:N]


# speedup vs baseline: 1.0994x; 1.0054x over previous
import functools
import jax
import jax.numpy as jnp
import numpy as np
from jax import lax
from jax.experimental import pallas as pl
from jax.experimental.pallas import tpu as pltpu
from jax.experimental.pallas import tpu_sc as plsc

N = 10000
IN_DIM = 128
HID_DIM = 512
OUT_DIM = 768
E = 320000

NC = 2
NS = 16
NW = NC * NS
B = 128
NB = -(-E // (B * NW))
NB = ((NB + 3) // 4) * 4
TB = NB * NW
EP = TB * B
IDX_LEN = EP + 8 * B
NPAD = 10112
ROWS_PER_TILE = NPAD // NS

BN = 2048
GRID = -(-NPAD // BN)

_AR = np.arange(IDX_LEN - E, dtype=np.int32)
_PAD_SRC = (N + _AR % (NPAD - N)).astype(np.int32)
_PAD_DST = (_AR % NPAD).astype(np.int32)
_PAD_DEG = (N + _AR % (NPAD - N)).astype(np.int32)

@functools.cache
def _mesh():
    return plsc.VectorSubcoreMesh(core_axis_name="c", subcore_axis_name="s",
                                  num_cores=NC, num_subcores=NS)


def _deg_body(zeros_hbm, ones_hbm, dst_hbm, out_hbm, di0, di1, rows, acc,
              is0, is1):
    c = lax.axis_index("c")
    s = lax.axis_index("s")
    wid = s * NC + c
    r0 = s * ROWS_PER_TILE
    pltpu.sync_copy(zeros_hbm.at[pl.ds(r0, ROWS_PER_TILE)],
                    acc.at[pl.ds(r0, ROWS_PER_TILE)])
    base = wid * (NB * B)
    pltpu.async_copy(dst_hbm.at[pl.ds(base, B)], di0, is0)
    pltpu.async_copy(dst_hbm.at[pl.ds(base + B, B)], di1, is1)
    pltpu.sync_copy(ones_hbm, rows)
    plsc.subcore_barrier()

    @pl.loop(0, NB // 2)
    def _(j):
        b = base + 2 * j * B
        pltpu.make_async_copy(dst_hbm.at[pl.ds(0, B)], di0, is0).wait()
        pltpu.sync_copy(rows, acc.at[di0], add=True)
        pltpu.async_copy(dst_hbm.at[pl.ds(b + 2 * B, B)], di0, is0)
        pltpu.make_async_copy(dst_hbm.at[pl.ds(0, B)], di1, is1).wait()
        pltpu.sync_copy(rows, acc.at[di1], add=True)
        pltpu.async_copy(dst_hbm.at[pl.ds(b + 3 * B, B)], di1, is1)

    pltpu.make_async_copy(dst_hbm.at[pl.ds(0, B)], di0, is0).wait()
    pltpu.make_async_copy(dst_hbm.at[pl.ds(0, B)], di1, is1).wait()
    plsc.subcore_barrier()
    pltpu.sync_copy(acc.at[pl.ds(r0, ROWS_PER_TILE)],
                    out_hbm.at[c, pl.ds(r0, ROWS_PER_TILE)])


def _prop_body(zeros_hbm, src_hbm, dst_hbm, table_hbm, out_hbm,
               si0, di0, si1, di1, si2, di2, si3, di3, rows0, rows1, acc,
               gs0, gs1, is0, is1, is2, is3):
    c = lax.axis_index("c")
    s = lax.axis_index("s")
    wid = s * NC + c
    r0 = s * ROWS_PER_TILE
    pltpu.sync_copy(zeros_hbm.at[pl.ds(r0, ROWS_PER_TILE)],
                    acc.at[pl.ds(r0, ROWS_PER_TILE)])
    base = wid * NB * B
    pltpu.sync_copy(src_hbm.at[pl.ds(base, B)], si0)
    pltpu.sync_copy(dst_hbm.at[pl.ds(base, B)], di0)
    pltpu.sync_copy(src_hbm.at[pl.ds(base + B, B)], si1)
    pltpu.sync_copy(dst_hbm.at[pl.ds(base + B, B)], di1)
    pltpu.async_copy(src_hbm.at[pl.ds(base + 2 * B, B)], si2, is2)
    pltpu.async_copy(dst_hbm.at[pl.ds(base + 2 * B, B)], di2, is2)
    pltpu.async_copy(src_hbm.at[pl.ds(base + 3 * B, B)], si3, is3)
    pltpu.async_copy(dst_hbm.at[pl.ds(base + 3 * B, B)], di3, is3)
    plsc.subcore_barrier()
    pltpu.async_copy(table_hbm.at[si0], rows0, gs0)
    pltpu.async_copy(table_hbm.at[si1], rows1, gs1)

    @pl.loop(0, NB // 4)
    def _(j):
        b = base + 4 * j * B
        pltpu.make_async_copy(table_hbm.at[si0], rows0, gs0).wait()
        pltpu.sync_copy(rows0, acc.at[di0], add=True)
        pltpu.async_copy(src_hbm.at[pl.ds(b + 4 * B, B)], si0, is0)
        pltpu.async_copy(dst_hbm.at[pl.ds(b + 4 * B, B)], di0, is0)
        pltpu.make_async_copy(src_hbm.at[pl.ds(0, B)], si2, is2).wait()
        pltpu.make_async_copy(dst_hbm.at[pl.ds(0, B)], di2, is2).wait()
        pltpu.async_copy(table_hbm.at[si2], rows0, gs0)
        pltpu.make_async_copy(table_hbm.at[si1], rows1, gs1).wait()
        pltpu.sync_copy(rows1, acc.at[di1], add=True)
        pltpu.async_copy(src_hbm.at[pl.ds(b + 5 * B, B)], si1, is1)
        pltpu.async_copy(dst_hbm.at[pl.ds(b + 5 * B, B)], di1, is1)
        pltpu.make_async_copy(src_hbm.at[pl.ds(0, B)], si3, is3).wait()
        pltpu.make_async_copy(dst_hbm.at[pl.ds(0, B)], di3, is3).wait()
        pltpu.async_copy(table_hbm.at[si3], rows1, gs1)
        pltpu.make_async_copy(table_hbm.at[si2], rows0, gs0).wait()
        pltpu.sync_copy(rows0, acc.at[di2], add=True)
        pltpu.async_copy(src_hbm.at[pl.ds(b + 6 * B, B)], si2, is2)
        pltpu.async_copy(dst_hbm.at[pl.ds(b + 6 * B, B)], di2, is2)
        pltpu.make_async_copy(src_hbm.at[pl.ds(0, B)], si0, is0).wait()
        pltpu.make_async_copy(dst_hbm.at[pl.ds(0, B)], di0, is0).wait()
        pltpu.async_copy(table_hbm.at[si0], rows0, gs0)
        pltpu.make_async_copy(table_hbm.at[si1], rows1, gs1).wait()
        pltpu.sync_copy(rows1, acc.at[di3], add=True)
        pltpu.async_copy(src_hbm.at[pl.ds(b + 7 * B, B)], si3, is3)
        pltpu.async_copy(dst_hbm.at[pl.ds(b + 7 * B, B)], di3, is3)
        pltpu.make_async_copy(src_hbm.at[pl.ds(0, B)], si1, is1).wait()
        pltpu.make_async_copy(dst_hbm.at[pl.ds(0, B)], di1, is1).wait()
        pltpu.async_copy(table_hbm.at[si1], rows1, gs1)

    pltpu.make_async_copy(table_hbm.at[si0], rows0, gs0).wait()
    pltpu.make_async_copy(table_hbm.at[si1], rows1, gs1).wait()
    pltpu.make_async_copy(src_hbm.at[pl.ds(0, B)], si2, is2).wait()
    pltpu.make_async_copy(dst_hbm.at[pl.ds(0, B)], di2, is2).wait()
    pltpu.make_async_copy(src_hbm.at[pl.ds(0, B)], si3, is3).wait()
    pltpu.make_async_copy(dst_hbm.at[pl.ds(0, B)], di3, is3).wait()
    plsc.subcore_barrier()
    pltpu.sync_copy(acc.at[pl.ds(r0, ROWS_PER_TILE)],
                    out_hbm.at[c, pl.ds(r0, ROWS_PER_TILE)])


@functools.cache
def _make_deg(interpret=False):
    return pl.kernel(
        _deg_body,
        out_type=jax.ShapeDtypeStruct((NC, NPAD, IN_DIM), jnp.float32),
        mesh=_mesh(),
        scratch_types=[
            pltpu.VMEM((B,), jnp.int32),
            pltpu.VMEM((B,), jnp.int32),
            pltpu.VMEM((B, IN_DIM), jnp.float32),
            pltpu.VMEM_SHARED((NPAD, IN_DIM), jnp.float32),
            pltpu.SemaphoreType.DMA,
            pltpu.SemaphoreType.DMA,
        ],
        interpret=interpret,
    )


@functools.cache
def _make_prop(interpret=False):
    return pl.kernel(
        _prop_body,
        out_type=jax.ShapeDtypeStruct((NC, NPAD, IN_DIM), jnp.float32),
        mesh=_mesh(),
        scratch_types=(
            [pltpu.VMEM((B,), jnp.int32)] * 8
            + [pltpu.VMEM((B, IN_DIM), jnp.float32)] * 2
            + [pltpu.VMEM_SHARED((NPAD, IN_DIM), jnp.float32)]
            + [pltpu.SemaphoreType.DMA] * 6
        ),
        interpret=interpret,
    )


def _scale_body(deg2_ref, x_ref, y1_ref, dinv_ref):
    deg = deg2_ref[0, :, 0:1] + deg2_ref[1, :, 0:1] + 1.0
    dinv = lax.rsqrt(deg)
    y1_ref[...] = x_ref[...] * dinv
    dinv_ref[...] = jnp.broadcast_to(dinv, (BN, IN_DIM))


def _make_scale(interpret=False):
    return pl.pallas_call(
        _scale_body,
        grid=(GRID,),
        in_specs=[
            pl.BlockSpec((NC, BN, IN_DIM), lambda i: (0, i, 0)),
            pl.BlockSpec((BN, IN_DIM), lambda i: (i, 0)),
        ],
        out_specs=[
            pl.BlockSpec((BN, IN_DIM), lambda i: (i, 0)),
            pl.BlockSpec((BN, IN_DIM), lambda i: (i, 0)),
        ],
        out_shape=[
            jax.ShapeDtypeStruct((NPAD, IN_DIM), jnp.float32),
            jax.ShapeDtypeStruct((NPAD, IN_DIM), jnp.float32),
        ],
        interpret=interpret,
    )


def _mm1_body(acc_ref, y1_ref, dinv_ref, w1_ref, b1_ref, *y2_refs):
    dinv = dinv_ref[...]
    sx = (acc_ref[0] + acc_ref[1] + y1_ref[...]) * dinv
    h = jnp.dot(sx, w1_ref[...], preferred_element_type=jnp.float32,
                precision=lax.Precision.DEFAULT)
    h = jnp.maximum(h + b1_ref[...], 0.0)
    y2 = h * dinv[:, 0:1]
    for ck in range(4):
        y2_refs[ck][...] = y2[:, ck * IN_DIM:(ck + 1) * IN_DIM]


def _make_mm1(interpret=False):
    return pl.pallas_call(
        _mm1_body,
        grid=(GRID,),
        in_specs=[
            pl.BlockSpec((NC, BN, IN_DIM), lambda i: (0, i, 0)),
            pl.BlockSpec((BN, IN_DIM), lambda i: (i, 0)),
            pl.BlockSpec((BN, IN_DIM), lambda i: (i, 0)),
            pl.BlockSpec((IN_DIM, HID_DIM), lambda i: (0, 0)),
            pl.BlockSpec((1, HID_DIM), lambda i: (0, 0)),
        ],
        out_specs=[pl.BlockSpec((BN, IN_DIM), lambda i: (i, 0))] * 4,
        out_shape=[jax.ShapeDtypeStruct((NPAD, IN_DIM), jnp.float32)] * 4,
        interpret=interpret,
    )


def _mm2_body(a0_ref, a1_ref, a2_ref, a3_ref, y0_ref, y1c_ref, y2c_ref,
              y3_ref, dinv_ref, x_ref, w2_ref, wl_ref, b2_ref, bl_ref,
              out_ref):
    dinv = dinv_ref[...]
    accs = (a0_ref, a1_ref, a2_ref, a3_ref)
    ys = (y0_ref, y1c_ref, y2c_ref, y3_ref)
    h2 = jnp.broadcast_to(b2_ref[...], (BN, OUT_DIM))
    for ck in range(4):
        sx = (accs[ck][0] + accs[ck][1] + ys[ck][...]) * dinv
        h2 = h2 + jnp.dot(sx, w2_ref[ck * IN_DIM:(ck + 1) * IN_DIM, :],
                          preferred_element_type=jnp.float32,
                          precision=lax.Precision.DEFAULT)
    out = jnp.dot(h2, wl_ref[...], preferred_element_type=jnp.float32,
                  precision=lax.Precision.DEFAULT)
    out = jnp.maximum(out + bl_ref[...], 0.0)
    out_ref[:, 0:IN_DIM] = x_ref[...]
    out_ref[:, IN_DIM:] = out


def _make_mm2(interpret=False):
    return pl.pallas_call(
        _mm2_body,
        grid=(GRID,),
        in_specs=[
            pl.BlockSpec((NC, BN, IN_DIM), lambda i: (0, i, 0)),
            pl.BlockSpec((NC, BN, IN_DIM), lambda i: (0, i, 0)),
            pl.BlockSpec((NC, BN, IN_DIM), lambda i: (0, i, 0)),
            pl.BlockSpec((NC, BN, IN_DIM), lambda i: (0, i, 0)),
            pl.BlockSpec((BN, IN_DIM), lambda i: (i, 0)),
            pl.BlockSpec((BN, IN_DIM), lambda i: (i, 0)),
            pl.BlockSpec((BN, IN_DIM), lambda i: (i, 0)),
            pl.BlockSpec((BN, IN_DIM), lambda i: (i, 0)),
            pl.BlockSpec((BN, IN_DIM), lambda i: (i, 0)),
            pl.BlockSpec((BN, IN_DIM), lambda i: (i, 0)),
            pl.BlockSpec((HID_DIM, OUT_DIM), lambda i: (0, 0)),
            pl.BlockSpec((OUT_DIM, OUT_DIM), lambda i: (0, 0)),
            pl.BlockSpec((1, OUT_DIM), lambda i: (0, 0)),
            pl.BlockSpec((1, OUT_DIM), lambda i: (0, 0)),
        ],
        out_specs=pl.BlockSpec((BN, IN_DIM + OUT_DIM), lambda i: (i, 0)),
        out_shape=jax.ShapeDtypeStruct((N, IN_DIM + OUT_DIM), jnp.float32),
        interpret=interpret,
    )


_scale_k = _make_scale()
_mm1_k = _make_mm1()
_mm2_k = _make_mm2()


@jax.jit
def kernel(feature, edge_index, W1, b1, W2, b2, Wl, bl):
    _deg_k = _make_deg()
    _prop_k = _make_prop()
    src = edge_index[0].astype(jnp.int32)
    dst = edge_index[1].astype(jnp.int32)
    srcp = jnp.concatenate([src, jnp.asarray(_PAD_SRC)])
    dstp = jnp.concatenate([dst, jnp.asarray(_PAD_DST)])
    dstp_deg = jnp.concatenate([dst, jnp.asarray(_PAD_DEG)])
    xpad = jnp.pad(feature, ((0, NPAD - N), (0, 0)))
    zeros128 = jnp.zeros((NPAD, IN_DIM), jnp.float32)
    ones128 = jnp.ones((B, IN_DIM), jnp.float32)

    deg2 = _deg_k(zeros128, ones128, dstp_deg)
    y1, dinv128 = _scale_k(deg2, xpad)
    acc1 = _prop_k(zeros128, srcp, dstp, y1)
    y2_4 = _mm1_k(acc1, y1, dinv128, W1, b1.reshape(1, HID_DIM))
    accs = [_prop_k(zeros128, srcp, dstp, y2_4[ck]) for ck in range(4)]
    out = _mm2_k(accs[0], accs[1], accs[2], accs[3], *y2_4, dinv128, xpad,
                 W2, Wl, b2.reshape(1, OUT_DIM), bl.reshape(1, OUT_DIM))
    return out[:N]
